# Initial kernel scaffold; baseline (speedup 1.0000x reference)
import jax
import jax.numpy as jnp
from jax.experimental import pallas as pl


def kernel(x, edge_index, batch, W1, b1, g1, be1, W2, b2, g2, be2, W3, b3, g3, be3, W4, b4, g4, be4, fcW, fcb):
    raise NotImplementedError("write your pallas kernel here")


# speedup vs baseline: 12.0077x; 12.0077x over previous
import functools

import jax
import jax.numpy as jnp
from jax import lax
from jax.experimental import pallas as pl
from jax.experimental.pallas import tpu as pltpu
from jax.experimental.pallas import tpu_sc as plsc

N = 10000
E = 320000
F_IN = 128
H = 64
C = 10
G = 128
EPS = 1e-5

NPAD = 10112
ROWS_PER_TILE = NPAD // 16
CHUNK = 128
CPT = 79
EPAD = 32 * CHUNK * CPT
DEGW = 8

_mesh = plsc.VectorSubcoreMesh(core_axis_name="c", subcore_axis_name="s")


@functools.partial(
    pl.kernel,
    mesh=_mesh,
    out_type=jax.ShapeDtypeStruct((2, NPAD, H), jnp.float32),
    scratch_types=[
        pltpu.VMEM((CHUNK,), jnp.int32),
        pltpu.VMEM((CHUNK,), jnp.int32),
        pltpu.VMEM((CHUNK, H), jnp.float32),
        pltpu.VMEM_SHARED((NPAD, H), jnp.float32),
        pltpu.SemaphoreType.DMA,
    ],
    compiler_params=pltpu.CompilerParams(use_tc_tiling_on_sc=False),
)
def _sc_agg(src_hbm, dst_hbm, hp_hbm, zeros_hbm, out_hbm,
            src_v, dst_v, rows_v, acc_sh, sem):
    c = lax.axis_index("c")
    s = lax.axis_index("s")
    wid = c * 16 + s
    pltpu.sync_copy(zeros_hbm.at[pl.ds(s * ROWS_PER_TILE, ROWS_PER_TILE)],
                    acc_sh.at[pl.ds(s * ROWS_PER_TILE, ROWS_PER_TILE)])
    plsc.subcore_barrier()

    base = wid * (CPT * CHUNK)

    def body(j, carry):
        off = base + j * CHUNK
        pltpu.sync_copy(src_hbm.at[pl.ds(off, CHUNK)], src_v)
        pltpu.sync_copy(dst_hbm.at[pl.ds(off, CHUNK)], dst_v)
        pltpu.async_copy(hp_hbm.at[src_v], rows_v, sem).wait()
        pltpu.sync_copy(rows_v, acc_sh.at[dst_v], add=True)
        return carry

    lax.fori_loop(0, CPT, body, 0)
    plsc.subcore_barrier()
    pltpu.sync_copy(acc_sh.at[pl.ds(s * ROWS_PER_TILE, ROWS_PER_TILE)],
                    out_hbm.at[c, pl.ds(s * ROWS_PER_TILE, ROWS_PER_TILE)])


@functools.partial(
    pl.kernel,
    mesh=_mesh,
    out_type=jax.ShapeDtypeStruct((2, NPAD, DEGW), jnp.float32),
    scratch_types=[
        pltpu.VMEM((CHUNK,), jnp.int32),
        pltpu.VMEM((CHUNK, DEGW), jnp.float32),
        pltpu.VMEM_SHARED((NPAD, DEGW), jnp.float32),
    ],
    compiler_params=pltpu.CompilerParams(use_tc_tiling_on_sc=False),
)
def _sc_deg(dst_hbm, ones_hbm, zeros_hbm, out_hbm, dst_v, ones_v, acc_sh):
    c = lax.axis_index("c")
    s = lax.axis_index("s")
    wid = c * 16 + s
    pltpu.sync_copy(ones_hbm, ones_v)
    pltpu.sync_copy(zeros_hbm.at[pl.ds(s * ROWS_PER_TILE, ROWS_PER_TILE)],
                    acc_sh.at[pl.ds(s * ROWS_PER_TILE, ROWS_PER_TILE)])
    plsc.subcore_barrier()

    base = wid * (CPT * CHUNK)

    def body(j, carry):
        off = base + j * CHUNK
        pltpu.sync_copy(dst_hbm.at[pl.ds(off, CHUNK)], dst_v)
        pltpu.sync_copy(ones_v, acc_sh.at[dst_v], add=True)
        return carry

    lax.fori_loop(0, CPT, body, 0)
    plsc.subcore_barrier()
    pltpu.sync_copy(acc_sh.at[pl.ds(s * ROWS_PER_TILE, ROWS_PER_TILE)],
                    out_hbm.at[c, pl.ds(s * ROWS_PER_TILE, ROWS_PER_TILE)])


def _row_mask():
    rows = lax.broadcasted_iota(jnp.int32, (NPAD, 1), 0)
    return (rows < N).astype(jnp.float32)


def _tc_pre_body(degp_ref, x_ref, w1_ref, dinv_ref, hp_ref):
    mask = _row_mask()
    deg = degp_ref[0, :, 0:1] + degp_ref[1, :, 0:1] + mask
    dinv = jnp.where(deg > 0.0, lax.rsqrt(jnp.maximum(deg, 1e-30)), 0.0)
    dinv_ref[...] = dinv
    h = jnp.dot(x_ref[...], w1_ref[...], preferred_element_type=jnp.float32)
    hp_ref[...] = h * dinv


_tc_pre = pl.pallas_call(
    _tc_pre_body,
    out_shape=(
        jax.ShapeDtypeStruct((NPAD, 1), jnp.float32),
        jax.ShapeDtypeStruct((NPAD, H), jnp.float32),
    ),
)


def _bn_relu(aggp_ref, hp_ref, dinv_ref, b_ref, g_ref, be_ref):
    mask = _row_mask()
    dinv = dinv_ref[...]
    z = dinv * (aggp_ref[0] + aggp_ref[1] + hp_ref[...]) + b_ref[...]
    mean = jnp.sum(z * mask, axis=0, keepdims=True) * (1.0 / N)
    zc = z - mean
    var = jnp.sum(mask * zc * zc, axis=0, keepdims=True) * (1.0 / N)
    zn = zc * lax.rsqrt(var + EPS)
    return jnp.maximum(g_ref[...] * zn + be_ref[...], 0.0) * mask


def _tc_mid_body(aggp_ref, hp_ref, dinv_ref, b_ref, g_ref, be_ref, wn_ref,
                 hpn_ref):
    a = _bn_relu(aggp_ref, hp_ref, dinv_ref, b_ref, g_ref, be_ref)
    hpn_ref[...] = jnp.dot(a, wn_ref[...],
                           preferred_element_type=jnp.float32) * dinv_ref[...]


_tc_mid = pl.pallas_call(
    _tc_mid_body,
    out_shape=jax.ShapeDtypeStruct((NPAD, H), jnp.float32),
)


def _tc_fin_body(aggp_ref, hp_ref, dinv_ref, b_ref, g_ref, be_ref,
                 batch_ref, fcw_ref, fcb_ref, out_ref):
    a = _bn_relu(aggp_ref, hp_ref, dinv_ref, b_ref, g_ref, be_ref)
    gids = lax.broadcasted_iota(jnp.int32, (G, NPAD), 0)
    onehot_t = (batch_ref[...] == gids).astype(jnp.float32)
    sums = jnp.dot(onehot_t, a, preferred_element_type=jnp.float32)
    counts = jnp.sum(onehot_t, axis=1, keepdims=True)
    pooled = sums / jnp.maximum(counts, 1.0)
    logits = jnp.dot(pooled, fcw_ref[...],
                     preferred_element_type=jnp.float32) + fcb_ref[...]
    m = jnp.max(logits, axis=-1, keepdims=True)
    lse = m + jnp.log(jnp.sum(jnp.exp(logits - m), axis=-1, keepdims=True))
    out_ref[...] = logits - lse


_tc_fin = pl.pallas_call(
    _tc_fin_body,
    out_shape=jax.ShapeDtypeStruct((G, C), jnp.float32),
)


def kernel(x, edge_index, batch, W1, b1, g1, be1, W2, b2, g2, be2,
           W3, b3, g3, be3, W4, b4, g4, be4, fcW, fcb):
    pad = jnp.full((EPAD - E,), N, jnp.int32)
    src_p = jnp.concatenate([edge_index[0], pad])
    dst_p = jnp.concatenate([edge_index[1], pad])
    x_p = jnp.zeros((NPAD, F_IN), jnp.float32).at[:N].set(x)
    batch_p = jnp.full((NPAD,), G, jnp.int32).at[:N].set(batch).reshape(1, NPAD)
    zeros_h = jnp.zeros((NPAD, H), jnp.float32)
    zeros_d = jnp.zeros((NPAD, DEGW), jnp.float32)
    ones_d = jnp.ones((CHUNK, DEGW), jnp.float32)

    degp = _sc_deg(dst_p, ones_d, zeros_d)
    dinv, hp = _tc_pre(degp, x_p, W1)

    for (Wn, b, g, be) in ((W2, b2, g2, be2), (W3, b3, g3, be3),
                           (W4, b4, g4, be4)):
        aggp = _sc_agg(src_p, dst_p, hp, zeros_h)
        hp = _tc_mid(aggp, hp, dinv, b.reshape(1, H), g.reshape(1, H),
                     be.reshape(1, H), Wn)

    aggp = _sc_agg(src_p, dst_p, hp, zeros_h)
    out = _tc_fin(aggp, hp, dinv, b4.reshape(1, H), g4.reshape(1, H),
                  be4.reshape(1, H), batch_p, fcW, fcb.reshape(1, C))
    return out


# speedup vs baseline: 13.0530x; 1.0870x over previous
import functools

import jax
import jax.numpy as jnp
from jax import lax
from jax.experimental import pallas as pl
from jax.experimental.pallas import tpu as pltpu
from jax.experimental.pallas import tpu_sc as plsc

N = 10000
E = 320000
F_IN = 128
H = 64
C = 10
G = 128
EPS = 1e-5

NPAD = 10112
ROWS_PER_TILE = NPAD // 16
CHUNK = 128
CPT = 80
NB = 4
EPAD = 32 * CHUNK * CPT
DEGW = 8

_mesh = plsc.VectorSubcoreMesh(core_axis_name="c", subcore_axis_name="s")


@functools.partial(
    pl.kernel,
    mesh=_mesh,
    out_type=jax.ShapeDtypeStruct((2, NPAD, H), jnp.float32),
    scratch_types=[
        pltpu.VMEM((CPT, CHUNK), jnp.int32),
        pltpu.VMEM((CPT, CHUNK), jnp.int32),
        pltpu.VMEM((NB, CHUNK, H), jnp.float32),
        pltpu.VMEM_SHARED((NPAD, H), jnp.float32),
    ] + [pltpu.SemaphoreType.DMA] * NB,
    compiler_params=pltpu.CompilerParams(use_tc_tiling_on_sc=False),
)
def _sc_agg(src_hbm, dst_hbm, hp_hbm, zeros_hbm, out_hbm,
            src_v, dst_v, rows, acc_sh, *sems):
    c = lax.axis_index("c")
    s = lax.axis_index("s")
    wid = c * 16 + s
    pltpu.sync_copy(src_hbm.at[wid], src_v)
    pltpu.sync_copy(dst_hbm.at[wid], dst_v)
    for b in range(NB):
        pltpu.async_copy(hp_hbm.at[src_v.at[b]], rows.at[b], sems[b])
    pltpu.sync_copy(zeros_hbm.at[pl.ds(s * ROWS_PER_TILE, ROWS_PER_TILE)],
                    acc_sh.at[pl.ds(s * ROWS_PER_TILE, ROWS_PER_TILE)])
    plsc.subcore_barrier()

    def round_body(jj, carry):
        for b in range(NB):
            jb = jj * NB + b
            pltpu.make_async_copy(hp_hbm.at[pl.ds(0, CHUNK)], rows.at[b],
                                  sems[b]).wait()
            pltpu.sync_copy(rows.at[b], acc_sh.at[dst_v.at[jb]], add=True)

            @pl.when(jj + 1 < CPT // NB)
            def _():
                pltpu.async_copy(hp_hbm.at[src_v.at[jb + NB]], rows.at[b],
                                 sems[b])
        return carry

    lax.fori_loop(0, CPT // NB, round_body, 0)
    plsc.subcore_barrier()
    pltpu.sync_copy(acc_sh.at[pl.ds(s * ROWS_PER_TILE, ROWS_PER_TILE)],
                    out_hbm.at[c, pl.ds(s * ROWS_PER_TILE, ROWS_PER_TILE)])


@functools.partial(
    pl.kernel,
    mesh=_mesh,
    out_type=jax.ShapeDtypeStruct((2, NPAD, DEGW), jnp.float32),
    scratch_types=[
        pltpu.VMEM((CPT, CHUNK), jnp.int32),
        pltpu.VMEM((CHUNK, DEGW), jnp.float32),
        pltpu.VMEM_SHARED((NPAD, DEGW), jnp.float32),
    ],
    compiler_params=pltpu.CompilerParams(use_tc_tiling_on_sc=False),
)
def _sc_deg(dst_hbm, ones_hbm, zeros_hbm, out_hbm, dst_v, ones_v, acc_sh):
    c = lax.axis_index("c")
    s = lax.axis_index("s")
    wid = c * 16 + s
    pltpu.sync_copy(dst_hbm.at[wid], dst_v)
    pltpu.sync_copy(ones_hbm, ones_v)
    pltpu.sync_copy(zeros_hbm.at[pl.ds(s * ROWS_PER_TILE, ROWS_PER_TILE)],
                    acc_sh.at[pl.ds(s * ROWS_PER_TILE, ROWS_PER_TILE)])
    plsc.subcore_barrier()

    def body(j, carry):
        pltpu.sync_copy(ones_v, acc_sh.at[dst_v.at[j]], add=True)
        return carry

    lax.fori_loop(0, CPT, body, 0)
    plsc.subcore_barrier()
    pltpu.sync_copy(acc_sh.at[pl.ds(s * ROWS_PER_TILE, ROWS_PER_TILE)],
                    out_hbm.at[c, pl.ds(s * ROWS_PER_TILE, ROWS_PER_TILE)])


def _row_mask():
    rows = lax.broadcasted_iota(jnp.int32, (NPAD, 1), 0)
    return (rows < N).astype(jnp.float32)


def _tc_pre_body(degp_ref, x_ref, w1_ref, dinv_ref, hp_ref):
    mask = _row_mask()
    deg = degp_ref[0, :, 0:1] + degp_ref[1, :, 0:1] + mask
    dinv = jnp.where(deg > 0.0, lax.rsqrt(jnp.maximum(deg, 1e-30)), 0.0)
    dinv_ref[...] = dinv
    h = jnp.dot(x_ref[...], w1_ref[...], preferred_element_type=jnp.float32)
    hp_ref[...] = h * dinv


_tc_pre = pl.pallas_call(
    _tc_pre_body,
    out_shape=(
        jax.ShapeDtypeStruct((NPAD, 1), jnp.float32),
        jax.ShapeDtypeStruct((NPAD, H), jnp.float32),
    ),
)


def _bn_relu(aggp_ref, hp_ref, dinv_ref, b_ref, g_ref, be_ref):
    mask = _row_mask()
    dinv = dinv_ref[...]
    z = dinv * (aggp_ref[0] + aggp_ref[1] + hp_ref[...]) + b_ref[...]
    mean = jnp.sum(z * mask, axis=0, keepdims=True) * (1.0 / N)
    zc = z - mean
    var = jnp.sum(mask * zc * zc, axis=0, keepdims=True) * (1.0 / N)
    zn = zc * lax.rsqrt(var + EPS)
    return jnp.maximum(g_ref[...] * zn + be_ref[...], 0.0) * mask


def _tc_mid_body(aggp_ref, hp_ref, dinv_ref, b_ref, g_ref, be_ref, wn_ref,
                 hpn_ref):
    a = _bn_relu(aggp_ref, hp_ref, dinv_ref, b_ref, g_ref, be_ref)
    hpn_ref[...] = jnp.dot(a, wn_ref[...],
                           preferred_element_type=jnp.float32) * dinv_ref[...]


_tc_mid = pl.pallas_call(
    _tc_mid_body,
    out_shape=jax.ShapeDtypeStruct((NPAD, H), jnp.float32),
)


def _tc_fin_body(aggp_ref, hp_ref, dinv_ref, b_ref, g_ref, be_ref,
                 batch_ref, fcw_ref, fcb_ref, out_ref):
    a = _bn_relu(aggp_ref, hp_ref, dinv_ref, b_ref, g_ref, be_ref)
    gids = lax.broadcasted_iota(jnp.int32, (G, NPAD), 0)
    onehot_t = (batch_ref[...] == gids).astype(jnp.float32)
    sums = jnp.dot(onehot_t, a, preferred_element_type=jnp.float32)
    counts = jnp.sum(onehot_t, axis=1, keepdims=True)
    pooled = sums / jnp.maximum(counts, 1.0)
    logits = jnp.dot(pooled, fcw_ref[...],
                     preferred_element_type=jnp.float32) + fcb_ref[...]
    m = jnp.max(logits, axis=-1, keepdims=True)
    lse = m + jnp.log(jnp.sum(jnp.exp(logits - m), axis=-1, keepdims=True))
    out_ref[...] = logits - lse


_tc_fin = pl.pallas_call(
    _tc_fin_body,
    out_shape=jax.ShapeDtypeStruct((G, C), jnp.float32),
)


def kernel(x, edge_index, batch, W1, b1, g1, be1, W2, b2, g2, be2,
           W3, b3, g3, be3, W4, b4, g4, be4, fcW, fcb):
    pad = jnp.full((EPAD - E,), N, jnp.int32)
    src_p = jnp.concatenate([edge_index[0], pad]).reshape(32, CPT, CHUNK)
    dst_p = jnp.concatenate([edge_index[1], pad]).reshape(32, CPT, CHUNK)
    x_p = jnp.zeros((NPAD, F_IN), jnp.float32).at[:N].set(x)
    batch_p = jnp.full((NPAD,), G, jnp.int32).at[:N].set(batch).reshape(1, NPAD)
    zeros_h = jnp.zeros((NPAD, H), jnp.float32)
    zeros_d = jnp.zeros((NPAD, DEGW), jnp.float32)
    ones_d = jnp.ones((CHUNK, DEGW), jnp.float32)

    degp = _sc_deg(dst_p, ones_d, zeros_d)
    dinv, hp = _tc_pre(degp, x_p, W1)

    for (Wn, b, g, be) in ((W2, b2, g2, be2), (W3, b3, g3, be3),
                           (W4, b4, g4, be4)):
        aggp = _sc_agg(src_p, dst_p, hp, zeros_h)
        hp = _tc_mid(aggp, hp, dinv, b.reshape(1, H), g.reshape(1, H),
                     be.reshape(1, H), Wn)

    aggp = _sc_agg(src_p, dst_p, hp, zeros_h)
    out = _tc_fin(aggp, hp, dinv, b4.reshape(1, H), g4.reshape(1, H),
                  be4.reshape(1, H), batch_p, fcW, fcb.reshape(1, C))
    return out


# speedup vs baseline: 13.1180x; 1.0050x over previous
import functools

import jax
import jax.numpy as jnp
from jax import lax
from jax.experimental import pallas as pl
from jax.experimental.pallas import tpu as pltpu
from jax.experimental.pallas import tpu_sc as plsc

N = 10000
E = 320000
F_IN = 128
H = 64
C = 10
G = 128
EPS = 1e-5

NPAD = 10112
ROWS_PER_TILE = NPAD // 16
CHUNK = 128
CPT = 80
NB = 8
EPAD = 32 * CHUNK * CPT
DEGW = 8

_mesh = plsc.VectorSubcoreMesh(core_axis_name="c", subcore_axis_name="s")


@functools.partial(
    pl.kernel,
    mesh=_mesh,
    out_type=jax.ShapeDtypeStruct((2, NPAD, H), jnp.float32),
    scratch_types=[
        pltpu.VMEM((CPT, CHUNK), jnp.int32),
        pltpu.VMEM((CPT, CHUNK), jnp.int32),
        pltpu.VMEM((NB, CHUNK, H), jnp.float32),
        pltpu.VMEM_SHARED((NPAD, H), jnp.float32),
    ] + [pltpu.SemaphoreType.DMA] * NB,
    compiler_params=pltpu.CompilerParams(use_tc_tiling_on_sc=False),
)
def _sc_agg(src_hbm, dst_hbm, hp_hbm, zeros_hbm, out_hbm,
            src_v, dst_v, rows, acc_sh, *sems):
    c = lax.axis_index("c")
    s = lax.axis_index("s")
    wid = c * 16 + s
    pltpu.sync_copy(src_hbm.at[wid], src_v)
    pltpu.sync_copy(dst_hbm.at[wid], dst_v)
    for b in range(NB):
        pltpu.async_copy(hp_hbm.at[src_v.at[b]], rows.at[b], sems[b])
    pltpu.sync_copy(zeros_hbm.at[pl.ds(s * ROWS_PER_TILE, ROWS_PER_TILE)],
                    acc_sh.at[pl.ds(s * ROWS_PER_TILE, ROWS_PER_TILE)])
    plsc.subcore_barrier()

    def round_body(jj, carry):
        for b in range(NB):
            jb = jj * NB + b
            pltpu.make_async_copy(hp_hbm.at[pl.ds(0, CHUNK)], rows.at[b],
                                  sems[b]).wait()
            pltpu.sync_copy(rows.at[b], acc_sh.at[dst_v.at[jb]], add=True)

            @pl.when(jj + 1 < CPT // NB)
            def _():
                pltpu.async_copy(hp_hbm.at[src_v.at[jb + NB]], rows.at[b],
                                 sems[b])
        return carry

    lax.fori_loop(0, CPT // NB, round_body, 0)
    plsc.subcore_barrier()
    pltpu.sync_copy(acc_sh.at[pl.ds(s * ROWS_PER_TILE, ROWS_PER_TILE)],
                    out_hbm.at[c, pl.ds(s * ROWS_PER_TILE, ROWS_PER_TILE)])


@functools.partial(
    pl.kernel,
    mesh=_mesh,
    out_type=jax.ShapeDtypeStruct((2, NPAD, DEGW), jnp.float32),
    scratch_types=[
        pltpu.VMEM((CPT, CHUNK), jnp.int32),
        pltpu.VMEM((CHUNK, DEGW), jnp.float32),
        pltpu.VMEM_SHARED((NPAD, DEGW), jnp.float32),
    ],
    compiler_params=pltpu.CompilerParams(use_tc_tiling_on_sc=False),
)
def _sc_deg(dst_hbm, ones_hbm, zeros_hbm, out_hbm, dst_v, ones_v, acc_sh):
    c = lax.axis_index("c")
    s = lax.axis_index("s")
    wid = c * 16 + s
    pltpu.sync_copy(dst_hbm.at[wid], dst_v)
    pltpu.sync_copy(ones_hbm, ones_v)
    pltpu.sync_copy(zeros_hbm.at[pl.ds(s * ROWS_PER_TILE, ROWS_PER_TILE)],
                    acc_sh.at[pl.ds(s * ROWS_PER_TILE, ROWS_PER_TILE)])
    plsc.subcore_barrier()

    def body(j, carry):
        pltpu.sync_copy(ones_v, acc_sh.at[dst_v.at[j]], add=True)
        return carry

    lax.fori_loop(0, CPT, body, 0)
    plsc.subcore_barrier()
    pltpu.sync_copy(acc_sh.at[pl.ds(s * ROWS_PER_TILE, ROWS_PER_TILE)],
                    out_hbm.at[c, pl.ds(s * ROWS_PER_TILE, ROWS_PER_TILE)])


def _row_mask():
    rows = lax.broadcasted_iota(jnp.int32, (NPAD, 1), 0)
    return (rows < N).astype(jnp.float32)


def _tc_pre_body(degp_ref, x_ref, w1_ref, dinv_ref, hp_ref):
    mask = _row_mask()
    deg = degp_ref[0, :, 0:1] + degp_ref[1, :, 0:1] + mask
    dinv = jnp.where(deg > 0.0, lax.rsqrt(jnp.maximum(deg, 1e-30)), 0.0)
    dinv_ref[...] = dinv
    h = jnp.dot(x_ref[...], w1_ref[...], preferred_element_type=jnp.float32)
    hp_ref[...] = h * dinv


_tc_pre = pl.pallas_call(
    _tc_pre_body,
    out_shape=(
        jax.ShapeDtypeStruct((NPAD, 1), jnp.float32),
        jax.ShapeDtypeStruct((NPAD, H), jnp.float32),
    ),
)


def _bn_relu(aggp_ref, hp_ref, dinv_ref, b_ref, g_ref, be_ref):
    mask = _row_mask()
    dinv = dinv_ref[...]
    z = dinv * (aggp_ref[0] + aggp_ref[1] + hp_ref[...]) + b_ref[...]
    mean = jnp.sum(z * mask, axis=0, keepdims=True) * (1.0 / N)
    zc = z - mean
    var = jnp.sum(mask * zc * zc, axis=0, keepdims=True) * (1.0 / N)
    zn = zc * lax.rsqrt(var + EPS)
    return jnp.maximum(g_ref[...] * zn + be_ref[...], 0.0) * mask


def _tc_mid_body(aggp_ref, hp_ref, dinv_ref, b_ref, g_ref, be_ref, wn_ref,
                 hpn_ref):
    a = _bn_relu(aggp_ref, hp_ref, dinv_ref, b_ref, g_ref, be_ref)
    hpn_ref[...] = jnp.dot(a, wn_ref[...],
                           preferred_element_type=jnp.float32) * dinv_ref[...]


_tc_mid = pl.pallas_call(
    _tc_mid_body,
    out_shape=jax.ShapeDtypeStruct((NPAD, H), jnp.float32),
)


def _tc_fin_body(aggp_ref, hp_ref, dinv_ref, b_ref, g_ref, be_ref,
                 batch_ref, fcw_ref, fcb_ref, out_ref):
    a = _bn_relu(aggp_ref, hp_ref, dinv_ref, b_ref, g_ref, be_ref)
    gids = lax.broadcasted_iota(jnp.int32, (G, NPAD), 0)
    onehot_t = (batch_ref[...] == gids).astype(jnp.float32)
    sums = jnp.dot(onehot_t, a, preferred_element_type=jnp.float32)
    counts = jnp.sum(onehot_t, axis=1, keepdims=True)
    pooled = sums / jnp.maximum(counts, 1.0)
    logits = jnp.dot(pooled, fcw_ref[...],
                     preferred_element_type=jnp.float32) + fcb_ref[...]
    m = jnp.max(logits, axis=-1, keepdims=True)
    lse = m + jnp.log(jnp.sum(jnp.exp(logits - m), axis=-1, keepdims=True))
    out_ref[...] = logits - lse


_tc_fin = pl.pallas_call(
    _tc_fin_body,
    out_shape=jax.ShapeDtypeStruct((G, C), jnp.float32),
)


def kernel(x, edge_index, batch, W1, b1, g1, be1, W2, b2, g2, be2,
           W3, b3, g3, be3, W4, b4, g4, be4, fcW, fcb):
    pad = jnp.full((EPAD - E,), N, jnp.int32)
    src_p = jnp.concatenate([edge_index[0], pad]).reshape(32, CPT, CHUNK)
    dst_p = jnp.concatenate([edge_index[1], pad]).reshape(32, CPT, CHUNK)
    x_p = jnp.zeros((NPAD, F_IN), jnp.float32).at[:N].set(x)
    batch_p = jnp.full((NPAD,), G, jnp.int32).at[:N].set(batch).reshape(1, NPAD)
    zeros_h = jnp.zeros((NPAD, H), jnp.float32)
    zeros_d = jnp.zeros((NPAD, DEGW), jnp.float32)
    ones_d = jnp.ones((CHUNK, DEGW), jnp.float32)

    degp = _sc_deg(dst_p, ones_d, zeros_d)
    dinv, hp = _tc_pre(degp, x_p, W1)

    for (Wn, b, g, be) in ((W2, b2, g2, be2), (W3, b3, g3, be3),
                           (W4, b4, g4, be4)):
        aggp = _sc_agg(src_p, dst_p, hp, zeros_h)
        hp = _tc_mid(aggp, hp, dinv, b.reshape(1, H), g.reshape(1, H),
                     be.reshape(1, H), Wn)

    aggp = _sc_agg(src_p, dst_p, hp, zeros_h)
    out = _tc_fin(aggp, hp, dinv, b4.reshape(1, H), g4.reshape(1, H),
                  be4.reshape(1, H), batch_p, fcW, fcb.reshape(1, C))
    return out


# speedup vs baseline: 27.7274x; 2.1137x over previous
import functools

import jax
import jax.numpy as jnp
from jax import lax
from jax.experimental import pallas as pl
from jax.experimental.pallas import tpu as pltpu
from jax.experimental.pallas import tpu_sc as plsc

N = 10000
E = 320000
F_IN = 128
H = 64
HH = H // 2
C = 10
G = 128
EPS = 1e-5

NPAD = 10112
ROWS_PER_TILE = NPAD // 16
CHUNK = 128
CPT_A = 160
NB = 8
CPT_D = 80
EPAD = 16 * CPT_A * CHUNK
DEGW = 8

_mesh = plsc.VectorSubcoreMesh(core_axis_name="c", subcore_axis_name="s")


@functools.partial(
    pl.kernel,
    mesh=_mesh,
    out_type=jax.ShapeDtypeStruct((2, NPAD, HH), jnp.float32),
    scratch_types=[
        pltpu.VMEM((CPT_A, CHUNK), jnp.int32),
        pltpu.VMEM((CPT_A, CHUNK), jnp.int32),
        pltpu.VMEM((NB, CHUNK, HH), jnp.float32),
        pltpu.VMEM_SHARED((NPAD, HH), jnp.float32),
        pltpu.VMEM_SHARED((NPAD, HH), jnp.float32),
    ] + [pltpu.SemaphoreType.DMA] * NB,
    compiler_params=pltpu.CompilerParams(use_tc_tiling_on_sc=False),
)
def _sc_agg(src_hbm, dst_hbm, hp2_hbm, zeros_hbm, out_hbm,
            src_v, dst_v, rows, acc_sh, hp_sh, *sems):
    c = lax.axis_index("c")
    s = lax.axis_index("s")
    pltpu.sync_copy(src_hbm.at[s], src_v)
    pltpu.sync_copy(dst_hbm.at[s], dst_v)
    rsl = pl.ds(s * ROWS_PER_TILE, ROWS_PER_TILE)
    pltpu.sync_copy(hp2_hbm.at[c, rsl], hp_sh.at[rsl])
    pltpu.sync_copy(zeros_hbm.at[rsl], acc_sh.at[rsl])
    plsc.subcore_barrier()
    for b in range(NB):
        pltpu.async_copy(hp_sh.at[src_v.at[b]], rows.at[b], sems[b])

    def round_body(jj, carry):
        for b in range(NB):
            jb = jj * NB + b
            pltpu.make_async_copy(hp2_hbm.at[0, pl.ds(0, CHUNK)], rows.at[b],
                                  sems[b]).wait()
            pltpu.sync_copy(rows.at[b], acc_sh.at[dst_v.at[jb]], add=True)

            @pl.when(jj + 1 < CPT_A // NB)
            def _():
                pltpu.async_copy(hp_sh.at[src_v.at[jb + NB]], rows.at[b],
                                 sems[b])
        return carry

    lax.fori_loop(0, CPT_A // NB, round_body, 0)
    plsc.subcore_barrier()
    pltpu.sync_copy(acc_sh.at[rsl], out_hbm.at[c, rsl])


@functools.partial(
    pl.kernel,
    mesh=_mesh,
    out_type=jax.ShapeDtypeStruct((2, NPAD, DEGW), jnp.float32),
    scratch_types=[
        pltpu.VMEM((CPT_D, CHUNK), jnp.int32),
        pltpu.VMEM((CHUNK, DEGW), jnp.float32),
        pltpu.VMEM_SHARED((NPAD, DEGW), jnp.float32),
    ],
    compiler_params=pltpu.CompilerParams(use_tc_tiling_on_sc=False),
)
def _sc_deg(dst_hbm, ones_hbm, zeros_hbm, out_hbm, dst_v, ones_v, acc_sh):
    c = lax.axis_index("c")
    s = lax.axis_index("s")
    wid = c * 16 + s
    pltpu.sync_copy(dst_hbm.at[wid], dst_v)
    pltpu.sync_copy(ones_hbm, ones_v)
    rsl = pl.ds(s * ROWS_PER_TILE, ROWS_PER_TILE)
    pltpu.sync_copy(zeros_hbm.at[rsl], acc_sh.at[rsl])
    plsc.subcore_barrier()

    def body(j, carry):
        pltpu.sync_copy(ones_v, acc_sh.at[dst_v.at[j]], add=True)
        return carry

    lax.fori_loop(0, CPT_D, body, 0)
    plsc.subcore_barrier()
    pltpu.sync_copy(acc_sh.at[rsl], out_hbm.at[c, rsl])


def _row_mask():
    rows = lax.broadcasted_iota(jnp.int32, (NPAD, 1), 0)
    return (rows < N).astype(jnp.float32)


def _split_cols(h, out_ref):
    out_ref[0] = h[:, :HH]
    out_ref[1] = h[:, HH:]


def _tc_pre_body(degp_ref, x_ref, w1_ref, dinv_ref, hp2_ref):
    mask = _row_mask()
    deg = degp_ref[0, :, 0:1] + degp_ref[1, :, 0:1] + mask
    dinv = jnp.where(deg > 0.0, lax.rsqrt(jnp.maximum(deg, 1e-30)), 0.0)
    dinv_ref[...] = dinv
    h = jnp.dot(x_ref[...], w1_ref[...], preferred_element_type=jnp.float32)
    _split_cols(h * dinv, hp2_ref)


_tc_pre = pl.pallas_call(
    _tc_pre_body,
    out_shape=(
        jax.ShapeDtypeStruct((NPAD, 1), jnp.float32),
        jax.ShapeDtypeStruct((2, NPAD, HH), jnp.float32),
    ),
)


def _bn_relu(aggp_ref, hp2_ref, dinv_ref, b_ref, g_ref, be_ref):
    mask = _row_mask()
    dinv = dinv_ref[...]
    agg = jnp.concatenate([aggp_ref[0], aggp_ref[1]], axis=1)
    hp = jnp.concatenate([hp2_ref[0], hp2_ref[1]], axis=1)
    z = dinv * (agg + hp) + b_ref[...]
    mean = jnp.sum(z * mask, axis=0, keepdims=True) * (1.0 / N)
    zc = z - mean
    var = jnp.sum(mask * zc * zc, axis=0, keepdims=True) * (1.0 / N)
    zn = zc * lax.rsqrt(var + EPS)
    return jnp.maximum(g_ref[...] * zn + be_ref[...], 0.0) * mask


def _tc_mid_body(aggp_ref, hp2_ref, dinv_ref, b_ref, g_ref, be_ref, wn_ref,
                 hpn2_ref):
    a = _bn_relu(aggp_ref, hp2_ref, dinv_ref, b_ref, g_ref, be_ref)
    hn = jnp.dot(a, wn_ref[...], preferred_element_type=jnp.float32)
    _split_cols(hn * dinv_ref[...], hpn2_ref)


_tc_mid = pl.pallas_call(
    _tc_mid_body,
    out_shape=jax.ShapeDtypeStruct((2, NPAD, HH), jnp.float32),
)


def _tc_fin_body(aggp_ref, hp2_ref, dinv_ref, b_ref, g_ref, be_ref,
                 batch_ref, fcw_ref, fcb_ref, out_ref):
    a = _bn_relu(aggp_ref, hp2_ref, dinv_ref, b_ref, g_ref, be_ref)
    gids = lax.broadcasted_iota(jnp.int32, (G, NPAD), 0)
    onehot_t = (batch_ref[...] == gids).astype(jnp.float32)
    sums = jnp.dot(onehot_t, a, preferred_element_type=jnp.float32)
    counts = jnp.sum(onehot_t, axis=1, keepdims=True)
    pooled = sums / jnp.maximum(counts, 1.0)
    logits = jnp.dot(pooled, fcw_ref[...],
                     preferred_element_type=jnp.float32) + fcb_ref[...]
    m = jnp.max(logits, axis=-1, keepdims=True)
    lse = m + jnp.log(jnp.sum(jnp.exp(logits - m), axis=-1, keepdims=True))
    out_ref[...] = logits - lse


_tc_fin = pl.pallas_call(
    _tc_fin_body,
    out_shape=jax.ShapeDtypeStruct((G, C), jnp.float32),
)


def kernel(x, edge_index, batch, W1, b1, g1, be1, W2, b2, g2, be2,
           W3, b3, g3, be3, W4, b4, g4, be4, fcW, fcb):
    pad = jnp.full((EPAD - E,), N, jnp.int32)
    src_flat = jnp.concatenate([edge_index[0], pad])
    dst_flat = jnp.concatenate([edge_index[1], pad])
    src_a = src_flat.reshape(16, CPT_A, CHUNK)
    dst_a = dst_flat.reshape(16, CPT_A, CHUNK)
    dst_d = dst_flat.reshape(32, CPT_D, CHUNK)
    x_p = jnp.zeros((NPAD, F_IN), jnp.float32).at[:N].set(x)
    batch_p = jnp.full((NPAD,), G, jnp.int32).at[:N].set(batch).reshape(1, NPAD)
    zeros_hh = jnp.zeros((NPAD, HH), jnp.float32)
    zeros_d = jnp.zeros((NPAD, DEGW), jnp.float32)
    ones_d = jnp.ones((CHUNK, DEGW), jnp.float32)

    degp = _sc_deg(dst_d, ones_d, zeros_d)
    dinv, hp2 = _tc_pre(degp, x_p, W1)

    for (Wn, b, g, be) in ((W2, b2, g2, be2), (W3, b3, g3, be3),
                           (W4, b4, g4, be4)):
        aggp = _sc_agg(src_a, dst_a, hp2, zeros_hh)
        hp2 = _tc_mid(aggp, hp2, dinv, b.reshape(1, H), g.reshape(1, H),
                      be.reshape(1, H), Wn)

    aggp = _sc_agg(src_a, dst_a, hp2, zeros_hh)
    out = _tc_fin(aggp, hp2, dinv, b4.reshape(1, H), g4.reshape(1, H),
                  be4.reshape(1, H), batch_p, fcW, fcb.reshape(1, C))
    return out


# speedup vs baseline: 29.7985x; 1.0747x over previous
import functools

import jax
import jax.numpy as jnp
from jax import lax
from jax.experimental import pallas as pl
from jax.experimental.pallas import tpu as pltpu
from jax.experimental.pallas import tpu_sc as plsc

N = 10000
E = 320000
F_IN = 128
H = 64
HH = H // 2
C = 10
G = 128
EPS = 1e-5

NPAD = 10112
ROWS_PER_TILE = NPAD // 16
CHUNK = 128
CPT_A = 160
NB = 8
CPT_D = 80
EPAD = 16 * CPT_A * CHUNK
DEGW = 8

_mesh = plsc.VectorSubcoreMesh(core_axis_name="c", subcore_axis_name="s")


@functools.partial(
    pl.kernel,
    mesh=_mesh,
    out_type=jax.ShapeDtypeStruct((2, NPAD, HH), jnp.float32),
    scratch_types=[
        pltpu.VMEM((CPT_A, CHUNK), jnp.int32),
        pltpu.VMEM((CPT_A, CHUNK), jnp.int32),
        pltpu.VMEM((NB, CHUNK, HH), jnp.float32),
        pltpu.VMEM_SHARED((NPAD, HH), jnp.float32),
        pltpu.VMEM_SHARED((NPAD, HH), jnp.float32),
    ] + [pltpu.SemaphoreType.DMA] * (2 * NB),
    compiler_params=pltpu.CompilerParams(use_tc_tiling_on_sc=False),
)
def _sc_agg(src_hbm, dst_hbm, hp2_hbm, zeros_hbm, out_hbm,
            src_v, dst_v, rows, acc_sh, hp_sh, *sems):
    gsem = sems[:NB]
    ssem = sems[NB:]
    LG = NB // 2
    c = lax.axis_index("c")
    s = lax.axis_index("s")
    pltpu.sync_copy(src_hbm.at[s], src_v)
    pltpu.sync_copy(dst_hbm.at[s], dst_v)
    rsl = pl.ds(s * ROWS_PER_TILE, ROWS_PER_TILE)
    pltpu.sync_copy(hp2_hbm.at[c, rsl], hp_sh.at[rsl])
    pltpu.sync_copy(zeros_hbm.at[rsl], acc_sh.at[rsl])
    plsc.subcore_barrier()

    def _drain(sem, buf):
        pltpu.make_async_copy(hp2_hbm.at[0, pl.ds(0, CHUNK)], buf, sem).wait()

    for b in range(LG):
        pltpu.async_copy(hp_sh.at[src_v.at[b]], rows.at[b], gsem[b])

    for b in range(NB):
        _drain(gsem[b], rows.at[b])
        pltpu.async_copy(rows.at[b], acc_sh.at[dst_v.at[b]], ssem[b],
                         add=True)
        bg = (b + LG) % NB
        if b >= LG:
            _drain(ssem[bg], rows.at[bg])
        pltpu.async_copy(hp_sh.at[src_v.at[b + LG]], rows.at[bg], gsem[bg])

    def round_body(jj, carry):
        for b in range(NB):
            jb = jj * NB + b
            _drain(gsem[b], rows.at[b])
            pltpu.async_copy(rows.at[b], acc_sh.at[dst_v.at[jb]], ssem[b],
                             add=True)
            bg = (b + LG) % NB
            _drain(ssem[bg], rows.at[bg])

            @pl.when(jb + LG < CPT_A)
            def _():
                pltpu.async_copy(hp_sh.at[src_v.at[jb + LG]], rows.at[bg],
                                 gsem[bg])
        return carry

    lax.fori_loop(1, CPT_A // NB, round_body, 0)
    for b in range(LG, NB):
        _drain(ssem[b], rows.at[b])
    plsc.subcore_barrier()
    pltpu.sync_copy(acc_sh.at[rsl], out_hbm.at[c, rsl])


@functools.partial(
    pl.kernel,
    mesh=_mesh,
    out_type=jax.ShapeDtypeStruct((2, NPAD, DEGW), jnp.float32),
    scratch_types=[
        pltpu.VMEM((CPT_D, CHUNK), jnp.int32),
        pltpu.VMEM((CHUNK, DEGW), jnp.float32),
        pltpu.VMEM_SHARED((NPAD, DEGW), jnp.float32),
    ],
    compiler_params=pltpu.CompilerParams(use_tc_tiling_on_sc=False),
)
def _sc_deg(dst_hbm, ones_hbm, zeros_hbm, out_hbm, dst_v, ones_v, acc_sh):
    c = lax.axis_index("c")
    s = lax.axis_index("s")
    wid = c * 16 + s
    pltpu.sync_copy(dst_hbm.at[wid], dst_v)
    pltpu.sync_copy(ones_hbm, ones_v)
    rsl = pl.ds(s * ROWS_PER_TILE, ROWS_PER_TILE)
    pltpu.sync_copy(zeros_hbm.at[rsl], acc_sh.at[rsl])
    plsc.subcore_barrier()

    def body(j, carry):
        pltpu.sync_copy(ones_v, acc_sh.at[dst_v.at[j]], add=True)
        return carry

    lax.fori_loop(0, CPT_D, body, 0)
    plsc.subcore_barrier()
    pltpu.sync_copy(acc_sh.at[rsl], out_hbm.at[c, rsl])


def _row_mask():
    rows = lax.broadcasted_iota(jnp.int32, (NPAD, 1), 0)
    return (rows < N).astype(jnp.float32)


def _split_cols(h, out_ref):
    out_ref[0] = h[:, :HH]
    out_ref[1] = h[:, HH:]


def _tc_pre_body(degp_ref, x_ref, w1_ref, dinv_ref, hp2_ref):
    mask = _row_mask()
    deg = degp_ref[0, :, 0:1] + degp_ref[1, :, 0:1] + mask
    dinv = jnp.where(deg > 0.0, lax.rsqrt(jnp.maximum(deg, 1e-30)), 0.0)
    dinv_ref[...] = dinv
    h = jnp.dot(x_ref[...], w1_ref[...], preferred_element_type=jnp.float32)
    _split_cols(h * dinv, hp2_ref)


_tc_pre = pl.pallas_call(
    _tc_pre_body,
    out_shape=(
        jax.ShapeDtypeStruct((NPAD, 1), jnp.float32),
        jax.ShapeDtypeStruct((2, NPAD, HH), jnp.float32),
    ),
)


def _bn_relu(aggp_ref, hp2_ref, dinv_ref, b_ref, g_ref, be_ref):
    mask = _row_mask()
    dinv = dinv_ref[...]
    agg = jnp.concatenate([aggp_ref[0], aggp_ref[1]], axis=1)
    hp = jnp.concatenate([hp2_ref[0], hp2_ref[1]], axis=1)
    z = dinv * (agg + hp) + b_ref[...]
    mean = jnp.sum(z * mask, axis=0, keepdims=True) * (1.0 / N)
    zc = z - mean
    var = jnp.sum(mask * zc * zc, axis=0, keepdims=True) * (1.0 / N)
    zn = zc * lax.rsqrt(var + EPS)
    return jnp.maximum(g_ref[...] * zn + be_ref[...], 0.0) * mask


def _tc_mid_body(aggp_ref, hp2_ref, dinv_ref, b_ref, g_ref, be_ref, wn_ref,
                 hpn2_ref):
    a = _bn_relu(aggp_ref, hp2_ref, dinv_ref, b_ref, g_ref, be_ref)
    hn = jnp.dot(a, wn_ref[...], preferred_element_type=jnp.float32)
    _split_cols(hn * dinv_ref[...], hpn2_ref)


_tc_mid = pl.pallas_call(
    _tc_mid_body,
    out_shape=jax.ShapeDtypeStruct((2, NPAD, HH), jnp.float32),
)


def _tc_fin_body(aggp_ref, hp2_ref, dinv_ref, b_ref, g_ref, be_ref,
                 batch_ref, fcw_ref, fcb_ref, out_ref):
    a = _bn_relu(aggp_ref, hp2_ref, dinv_ref, b_ref, g_ref, be_ref)
    gids = lax.broadcasted_iota(jnp.int32, (G, NPAD), 0)
    onehot_t = (batch_ref[...] == gids).astype(jnp.float32)
    sums = jnp.dot(onehot_t, a, preferred_element_type=jnp.float32)
    counts = jnp.sum(onehot_t, axis=1, keepdims=True)
    pooled = sums / jnp.maximum(counts, 1.0)
    logits = jnp.dot(pooled, fcw_ref[...],
                     preferred_element_type=jnp.float32) + fcb_ref[...]
    m = jnp.max(logits, axis=-1, keepdims=True)
    lse = m + jnp.log(jnp.sum(jnp.exp(logits - m), axis=-1, keepdims=True))
    out_ref[...] = logits - lse


_tc_fin = pl.pallas_call(
    _tc_fin_body,
    out_shape=jax.ShapeDtypeStruct((G, C), jnp.float32),
)


def kernel(x, edge_index, batch, W1, b1, g1, be1, W2, b2, g2, be2,
           W3, b3, g3, be3, W4, b4, g4, be4, fcW, fcb):
    pad = jnp.full((EPAD - E,), N, jnp.int32)
    src_flat = jnp.concatenate([edge_index[0], pad])
    dst_flat = jnp.concatenate([edge_index[1], pad])
    src_a = src_flat.reshape(16, CPT_A, CHUNK)
    dst_a = dst_flat.reshape(16, CPT_A, CHUNK)
    dst_d = dst_flat.reshape(32, CPT_D, CHUNK)
    x_p = jnp.zeros((NPAD, F_IN), jnp.float32).at[:N].set(x)
    batch_p = jnp.full((NPAD,), G, jnp.int32).at[:N].set(batch).reshape(1, NPAD)
    zeros_hh = jnp.zeros((NPAD, HH), jnp.float32)
    zeros_d = jnp.zeros((NPAD, DEGW), jnp.float32)
    ones_d = jnp.ones((CHUNK, DEGW), jnp.float32)

    degp = _sc_deg(dst_d, ones_d, zeros_d)
    dinv, hp2 = _tc_pre(degp, x_p, W1)

    for (Wn, b, g, be) in ((W2, b2, g2, be2), (W3, b3, g3, be3),
                           (W4, b4, g4, be4)):
        aggp = _sc_agg(src_a, dst_a, hp2, zeros_hh)
        hp2 = _tc_mid(aggp, hp2, dinv, b.reshape(1, H), g.reshape(1, H),
                      be.reshape(1, H), Wn)

    aggp = _sc_agg(src_a, dst_a, hp2, zeros_hh)
    out = _tc_fin(aggp, hp2, dinv, b4.reshape(1, H), g4.reshape(1, H),
                  be4.reshape(1, H), batch_p, fcW, fcb.reshape(1, C))
    return out


# speedup vs baseline: 30.4089x; 1.0205x over previous
import functools

import jax
import jax.numpy as jnp
from jax import lax
from jax.experimental import pallas as pl
from jax.experimental.pallas import tpu as pltpu
from jax.experimental.pallas import tpu_sc as plsc

N = 10000
E = 320000
F_IN = 128
H = 64
HH = H // 2
C = 10
G = 128
EPS = 1e-5

NPAD = 10112
ROWS_PER_TILE = NPAD // 16
CHUNK = 128
CPT_A = 160
NB = 8
CPT_D = 80
EPAD = 16 * CPT_A * CHUNK
DEGW = 8

_mesh = plsc.VectorSubcoreMesh(core_axis_name="c", subcore_axis_name="s")


@functools.partial(
    pl.kernel,
    mesh=_mesh,
    out_type=jax.ShapeDtypeStruct((2, NPAD, HH), jnp.float32),
    scratch_types=[
        pltpu.VMEM((CPT_A, CHUNK), jnp.int32),
        pltpu.VMEM((CPT_A, CHUNK), jnp.int32),
        pltpu.VMEM((NB, CHUNK, HH), jnp.float32),
        pltpu.VMEM_SHARED((NPAD, HH), jnp.float32),
        pltpu.VMEM_SHARED((NPAD, HH), jnp.float32),
    ] + [pltpu.SemaphoreType.DMA] * (2 * NB),
    compiler_params=pltpu.CompilerParams(use_tc_tiling_on_sc=False),
)
def _sc_agg(src_hbm, dst_hbm, hp2_hbm, zeros_hbm, out_hbm,
            src_v, dst_v, rows, acc_sh, hp_sh, *sems):
    gsem = sems[:NB]
    ssem = sems[NB:]
    LG = NB // 2
    c = lax.axis_index("c")
    s = lax.axis_index("s")
    rsl = pl.ds(s * ROWS_PER_TILE, ROWS_PER_TILE)
    h_src = pltpu.async_copy(src_hbm.at[s], src_v, gsem[0])
    h_dst = pltpu.async_copy(dst_hbm.at[s], dst_v, gsem[1])
    h_hp = pltpu.async_copy(hp2_hbm.at[c, rsl], hp_sh.at[rsl], gsem[2])
    h_zero = pltpu.async_copy(zeros_hbm.at[rsl], acc_sh.at[rsl], gsem[3])
    h_src.wait()
    h_dst.wait()
    h_hp.wait()
    h_zero.wait()
    plsc.subcore_barrier()

    def _drain(sem, buf):
        pltpu.make_async_copy(hp2_hbm.at[0, pl.ds(0, CHUNK)], buf, sem).wait()

    for b in range(LG):
        pltpu.async_copy(hp_sh.at[src_v.at[b]], rows.at[b], gsem[b])

    for b in range(NB):
        _drain(gsem[b], rows.at[b])
        pltpu.async_copy(rows.at[b], acc_sh.at[dst_v.at[b]], ssem[b],
                         add=True)
        bg = (b + LG) % NB
        if b >= LG:
            _drain(ssem[bg], rows.at[bg])
        pltpu.async_copy(hp_sh.at[src_v.at[b + LG]], rows.at[bg], gsem[bg])

    def round_body(jj, carry):
        for b in range(NB):
            jb = jj * NB + b
            _drain(gsem[b], rows.at[b])
            pltpu.async_copy(rows.at[b], acc_sh.at[dst_v.at[jb]], ssem[b],
                             add=True)
            bg = (b + LG) % NB
            _drain(ssem[bg], rows.at[bg])

            @pl.when(jb + LG < CPT_A)
            def _():
                pltpu.async_copy(hp_sh.at[src_v.at[jb + LG]], rows.at[bg],
                                 gsem[bg])
        return carry

    lax.fori_loop(1, CPT_A // NB, round_body, 0)
    for b in range(LG, NB):
        _drain(ssem[b], rows.at[b])
    plsc.subcore_barrier()
    pltpu.sync_copy(acc_sh.at[rsl], out_hbm.at[c, rsl])


@functools.partial(
    pl.kernel,
    mesh=_mesh,
    out_type=jax.ShapeDtypeStruct((2, NPAD, DEGW), jnp.float32),
    scratch_types=[
        pltpu.VMEM((CPT_D, CHUNK), jnp.int32),
        pltpu.VMEM((CHUNK, DEGW), jnp.float32),
        pltpu.VMEM_SHARED((NPAD, DEGW), jnp.float32),
        pltpu.SemaphoreType.DMA,
    ],
    compiler_params=pltpu.CompilerParams(use_tc_tiling_on_sc=False),
)
def _sc_deg(dst_hbm, ones_hbm, zeros_hbm, out_hbm, dst_v, ones_v, acc_sh,
            sem):
    c = lax.axis_index("c")
    s = lax.axis_index("s")
    wid = c * 16 + s
    pltpu.sync_copy(dst_hbm.at[wid], dst_v)
    pltpu.sync_copy(ones_hbm, ones_v)
    rsl = pl.ds(s * ROWS_PER_TILE, ROWS_PER_TILE)
    pltpu.sync_copy(zeros_hbm.at[rsl], acc_sh.at[rsl])
    plsc.subcore_barrier()

    def body(j, carry):
        pltpu.async_copy(ones_v, acc_sh.at[dst_v.at[j]], sem, add=True)
        return carry

    lax.fori_loop(0, CPT_D, body, 0)

    def drain(j, carry):
        pltpu.make_async_copy(ones_hbm, ones_v, sem).wait()
        return carry

    lax.fori_loop(0, CPT_D, drain, 0)
    plsc.subcore_barrier()
    pltpu.sync_copy(acc_sh.at[rsl], out_hbm.at[c, rsl])


def _row_mask():
    rows = lax.broadcasted_iota(jnp.int32, (NPAD, 1), 0)
    return (rows < N).astype(jnp.float32)


def _split_cols(h, out_ref):
    out_ref[0] = h[:, :HH]
    out_ref[1] = h[:, HH:]


def _tc_pre_body(degp_ref, x_ref, w1_ref, dinv_ref, hp2_ref):
    mask = _row_mask()
    deg = degp_ref[0, :, 0:1] + degp_ref[1, :, 0:1] + mask
    dinv = jnp.where(deg > 0.0, lax.rsqrt(jnp.maximum(deg, 1e-30)), 0.0)
    dinv_ref[...] = dinv
    h = jnp.dot(x_ref[...], w1_ref[...], preferred_element_type=jnp.float32)
    _split_cols(h * dinv, hp2_ref)


_tc_pre = pl.pallas_call(
    _tc_pre_body,
    out_shape=(
        jax.ShapeDtypeStruct((NPAD, 1), jnp.float32),
        jax.ShapeDtypeStruct((2, NPAD, HH), jnp.float32),
    ),
)


def _bn_relu(aggp_ref, hp2_ref, dinv_ref, b_ref, g_ref, be_ref):
    mask = _row_mask()
    dinv = dinv_ref[...]
    agg = jnp.concatenate([aggp_ref[0], aggp_ref[1]], axis=1)
    hp = jnp.concatenate([hp2_ref[0], hp2_ref[1]], axis=1)
    z = dinv * (agg + hp) + b_ref[...]
    mean = jnp.sum(z * mask, axis=0, keepdims=True) * (1.0 / N)
    zc = z - mean
    var = jnp.sum(mask * zc * zc, axis=0, keepdims=True) * (1.0 / N)
    zn = zc * lax.rsqrt(var + EPS)
    return jnp.maximum(g_ref[...] * zn + be_ref[...], 0.0) * mask


def _tc_mid_body(aggp_ref, hp2_ref, dinv_ref, b_ref, g_ref, be_ref, wn_ref,
                 hpn2_ref):
    a = _bn_relu(aggp_ref, hp2_ref, dinv_ref, b_ref, g_ref, be_ref)
    hn = jnp.dot(a, wn_ref[...], preferred_element_type=jnp.float32)
    _split_cols(hn * dinv_ref[...], hpn2_ref)


_tc_mid = pl.pallas_call(
    _tc_mid_body,
    out_shape=jax.ShapeDtypeStruct((2, NPAD, HH), jnp.float32),
)


def _tc_fin_body(aggp_ref, hp2_ref, dinv_ref, b_ref, g_ref, be_ref,
                 batch_ref, fcw_ref, fcb_ref, out_ref):
    a = _bn_relu(aggp_ref, hp2_ref, dinv_ref, b_ref, g_ref, be_ref)
    gids = lax.broadcasted_iota(jnp.int32, (G, NPAD), 0)
    onehot_t = (batch_ref[...] == gids).astype(jnp.float32)
    sums = jnp.dot(onehot_t, a, preferred_element_type=jnp.float32)
    counts = jnp.sum(onehot_t, axis=1, keepdims=True)
    pooled = sums / jnp.maximum(counts, 1.0)
    logits = jnp.dot(pooled, fcw_ref[...],
                     preferred_element_type=jnp.float32) + fcb_ref[...]
    m = jnp.max(logits, axis=-1, keepdims=True)
    lse = m + jnp.log(jnp.sum(jnp.exp(logits - m), axis=-1, keepdims=True))
    out_ref[...] = logits - lse


_tc_fin = pl.pallas_call(
    _tc_fin_body,
    out_shape=jax.ShapeDtypeStruct((G, C), jnp.float32),
)


def kernel(x, edge_index, batch, W1, b1, g1, be1, W2, b2, g2, be2,
           W3, b3, g3, be3, W4, b4, g4, be4, fcW, fcb):
    pad = jnp.full((EPAD - E,), N, jnp.int32)
    src_flat = jnp.concatenate([edge_index[0], pad])
    dst_flat = jnp.concatenate([edge_index[1], pad])
    src_a = src_flat.reshape(16, CPT_A, CHUNK)
    dst_a = dst_flat.reshape(16, CPT_A, CHUNK)
    dst_d = dst_flat.reshape(32, CPT_D, CHUNK)
    x_p = jnp.zeros((NPAD, F_IN), jnp.float32).at[:N].set(x)
    batch_p = jnp.full((NPAD,), G, jnp.int32).at[:N].set(batch).reshape(1, NPAD)
    zeros_hh = jnp.zeros((NPAD, HH), jnp.float32)
    zeros_d = jnp.zeros((NPAD, DEGW), jnp.float32)
    ones_d = jnp.ones((CHUNK, DEGW), jnp.float32)

    degp = _sc_deg(dst_d, ones_d, zeros_d)
    dinv, hp2 = _tc_pre(degp, x_p, W1)

    for (Wn, b, g, be) in ((W2, b2, g2, be2), (W3, b3, g3, be3),
                           (W4, b4, g4, be4)):
        aggp = _sc_agg(src_a, dst_a, hp2, zeros_hh)
        hp2 = _tc_mid(aggp, hp2, dinv, b.reshape(1, H), g.reshape(1, H),
                      be.reshape(1, H), Wn)

    aggp = _sc_agg(src_a, dst_a, hp2, zeros_hh)
    out = _tc_fin(aggp, hp2, dinv, b4.reshape(1, H), g4.reshape(1, H),
                  be4.reshape(1, H), batch_p, fcW, fcb.reshape(1, C))
    return out


# speedup vs baseline: 30.5273x; 1.0039x over previous
import functools

import jax
import jax.numpy as jnp
from jax import lax
from jax.experimental import pallas as pl
from jax.experimental.pallas import tpu as pltpu
from jax.experimental.pallas import tpu_sc as plsc

N = 10000
E = 320000
F_IN = 128
H = 64
HH = H // 2
C = 10
G = 128
EPS = 1e-5

NPAD = 10112
ROWS_PER_TILE = NPAD // 16
CHUNK = 128
CPT_A = 160
CPG = 2
SLOTS = 4
LGS = SLOTS // 2
PAIRS = CPT_A // CPG
CPT_D = 80
EPAD = 16 * CPT_A * CHUNK
DEGW = 8

_mesh = plsc.VectorSubcoreMesh(core_axis_name="c", subcore_axis_name="s")


@functools.partial(
    pl.kernel,
    mesh=_mesh,
    out_type=jax.ShapeDtypeStruct((2, NPAD, HH), jnp.float32),
    scratch_types=[
        pltpu.VMEM((CPT_A, CHUNK), jnp.int32),
        pltpu.VMEM((CPT_A, CHUNK), jnp.int32),
        pltpu.VMEM((SLOTS, CPG * CHUNK, HH), jnp.float32),
        pltpu.VMEM_SHARED((NPAD, HH), jnp.float32),
        pltpu.VMEM_SHARED((NPAD, HH), jnp.float32),
    ] + [pltpu.SemaphoreType.DMA] * (2 * SLOTS),
    compiler_params=pltpu.CompilerParams(use_tc_tiling_on_sc=False),
)
def _sc_agg(src_hbm, dst_hbm, hp2_hbm, zeros_hbm, out_hbm,
            src_v, dst_v, rows, acc_sh, hp_sh, *sems):
    gsem = sems[:SLOTS]
    ssem = sems[SLOTS:]
    c = lax.axis_index("c")
    s = lax.axis_index("s")
    rsl = pl.ds(s * ROWS_PER_TILE, ROWS_PER_TILE)
    h_src = pltpu.async_copy(src_hbm.at[s], src_v, gsem[0])
    h_dst = pltpu.async_copy(dst_hbm.at[s], dst_v, gsem[1])
    h_hp = pltpu.async_copy(hp2_hbm.at[c, rsl], hp_sh.at[rsl], gsem[2])
    h_zero = pltpu.async_copy(zeros_hbm.at[rsl], acc_sh.at[rsl], gsem[3])
    h_src.wait()
    h_dst.wait()
    h_hp.wait()
    h_zero.wait()
    plsc.subcore_barrier()

    def _drain(sem, buf):
        pltpu.make_async_copy(hp2_hbm.at[0, pl.ds(0, CPG * CHUNK)], buf,
                              sem).wait()

    def _gathers(p, b):
        for q in range(CPG):
            pltpu.async_copy(hp_sh.at[src_v.at[p * CPG + q]],
                             rows.at[b, pl.ds(q * CHUNK, CHUNK)], gsem[b])

    def _scatters(p, b):
        for q in range(CPG):
            pltpu.async_copy(rows.at[b, pl.ds(q * CHUNK, CHUNK)],
                             acc_sh.at[dst_v.at[p * CPG + q]], ssem[b],
                             add=True)

    for b in range(LGS):
        _gathers(b, b)

    for b in range(SLOTS):
        _drain(gsem[b], rows.at[b])
        _scatters(b, b)
        bg = (b + LGS) % SLOTS
        if b >= LGS:
            _drain(ssem[bg], rows.at[bg])
        _gathers(b + LGS, bg)

    def round_body(jj, carry):
        for b in range(SLOTS):
            p = jj * SLOTS + b
            _drain(gsem[b], rows.at[b])
            _scatters(p, b)
            bg = (b + LGS) % SLOTS
            _drain(ssem[bg], rows.at[bg])

            @pl.when(p + LGS < PAIRS)
            def _():
                _gathers(p + LGS, bg)
        return carry

    lax.fori_loop(1, PAIRS // SLOTS, round_body, 0)
    for b in range(LGS, SLOTS):
        _drain(ssem[b], rows.at[b])
    plsc.subcore_barrier()
    pltpu.sync_copy(acc_sh.at[rsl], out_hbm.at[c, rsl])


@functools.partial(
    pl.kernel,
    mesh=_mesh,
    out_type=jax.ShapeDtypeStruct((2, NPAD, DEGW), jnp.float32),
    scratch_types=[
        pltpu.VMEM((CPT_D, CHUNK), jnp.int32),
        pltpu.VMEM((CHUNK, DEGW), jnp.float32),
        pltpu.VMEM_SHARED((NPAD, DEGW), jnp.float32),
        pltpu.SemaphoreType.DMA,
    ],
    compiler_params=pltpu.CompilerParams(use_tc_tiling_on_sc=False),
)
def _sc_deg(dst_hbm, ones_hbm, zeros_hbm, out_hbm, dst_v, ones_v, acc_sh,
            sem):
    c = lax.axis_index("c")
    s = lax.axis_index("s")
    wid = c * 16 + s
    pltpu.sync_copy(dst_hbm.at[wid], dst_v)
    pltpu.sync_copy(ones_hbm, ones_v)
    rsl = pl.ds(s * ROWS_PER_TILE, ROWS_PER_TILE)
    pltpu.sync_copy(zeros_hbm.at[rsl], acc_sh.at[rsl])
    plsc.subcore_barrier()

    def body(j, carry):
        pltpu.async_copy(ones_v, acc_sh.at[dst_v.at[j]], sem, add=True)
        return carry

    lax.fori_loop(0, CPT_D, body, 0)

    def drain(j, carry):
        pltpu.make_async_copy(ones_hbm, ones_v, sem).wait()
        return carry

    lax.fori_loop(0, CPT_D, drain, 0)
    plsc.subcore_barrier()
    pltpu.sync_copy(acc_sh.at[rsl], out_hbm.at[c, rsl])


def _row_mask():
    rows = lax.broadcasted_iota(jnp.int32, (NPAD, 1), 0)
    return (rows < N).astype(jnp.float32)


def _split_cols(h, out_ref):
    out_ref[0] = h[:, :HH]
    out_ref[1] = h[:, HH:]


def _tc_pre_body(degp_ref, x_ref, w1_ref, dinv_ref, hp2_ref):
    mask = _row_mask()
    deg = degp_ref[0, :, 0:1] + degp_ref[1, :, 0:1] + mask
    dinv = jnp.where(deg > 0.0, lax.rsqrt(jnp.maximum(deg, 1e-30)), 0.0)
    dinv_ref[...] = dinv
    h = jnp.dot(x_ref[...], w1_ref[...], preferred_element_type=jnp.float32)
    hp = h * dinv[:N]
    zpad = jnp.zeros((NPAD - N, HH), jnp.float32)
    hp2_ref[0] = jnp.concatenate([hp[:, :HH], zpad], axis=0)
    hp2_ref[1] = jnp.concatenate([hp[:, HH:], zpad], axis=0)


_tc_pre = pl.pallas_call(
    _tc_pre_body,
    out_shape=(
        jax.ShapeDtypeStruct((NPAD, 1), jnp.float32),
        jax.ShapeDtypeStruct((2, NPAD, HH), jnp.float32),
    ),
)


def _bn_relu(aggp_ref, hp2_ref, dinv_ref, b_ref, g_ref, be_ref):
    mask = _row_mask()
    dinv = dinv_ref[...]
    agg = jnp.concatenate([aggp_ref[0], aggp_ref[1]], axis=1)
    hp = jnp.concatenate([hp2_ref[0], hp2_ref[1]], axis=1)
    z = dinv * (agg + hp) + b_ref[...]
    mean = jnp.sum(z * mask, axis=0, keepdims=True) * (1.0 / N)
    zc = z - mean
    var = jnp.sum(mask * zc * zc, axis=0, keepdims=True) * (1.0 / N)
    zn = zc * lax.rsqrt(var + EPS)
    return jnp.maximum(g_ref[...] * zn + be_ref[...], 0.0) * mask


def _tc_mid_body(aggp_ref, hp2_ref, dinv_ref, b_ref, g_ref, be_ref, wn_ref,
                 hpn2_ref):
    a = _bn_relu(aggp_ref, hp2_ref, dinv_ref, b_ref, g_ref, be_ref)
    hn = jnp.dot(a, wn_ref[...], preferred_element_type=jnp.float32)
    _split_cols(hn * dinv_ref[...], hpn2_ref)


_tc_mid = pl.pallas_call(
    _tc_mid_body,
    out_shape=jax.ShapeDtypeStruct((2, NPAD, HH), jnp.float32),
)


def _tc_fin_body(aggp_ref, hp2_ref, dinv_ref, b_ref, g_ref, be_ref,
                 batch_ref, fcw_ref, fcb_ref, out_ref):
    a = _bn_relu(aggp_ref, hp2_ref, dinv_ref, b_ref, g_ref, be_ref)
    gids = lax.broadcasted_iota(jnp.int32, (G, NPAD), 0)
    onehot_t = (batch_ref[...] == gids).astype(jnp.float32)
    sums = jnp.dot(onehot_t, a, preferred_element_type=jnp.float32)
    counts = jnp.sum(onehot_t, axis=1, keepdims=True)
    pooled = sums / jnp.maximum(counts, 1.0)
    logits = jnp.dot(pooled, fcw_ref[...],
                     preferred_element_type=jnp.float32) + fcb_ref[...]
    m = jnp.max(logits, axis=-1, keepdims=True)
    lse = m + jnp.log(jnp.sum(jnp.exp(logits - m), axis=-1, keepdims=True))
    out_ref[...] = logits - lse


_tc_fin = pl.pallas_call(
    _tc_fin_body,
    out_shape=jax.ShapeDtypeStruct((G, C), jnp.float32),
)


def kernel(x, edge_index, batch, W1, b1, g1, be1, W2, b2, g2, be2,
           W3, b3, g3, be3, W4, b4, g4, be4, fcW, fcb):
    pad = jnp.full((EPAD - E,), N, jnp.int32)
    src_flat = jnp.concatenate([edge_index[0], pad])
    dst_flat = jnp.concatenate([edge_index[1], pad])
    src_a = src_flat.reshape(16, CPT_A, CHUNK)
    dst_a = dst_flat.reshape(16, CPT_A, CHUNK)
    dst_d = dst_flat.reshape(32, CPT_D, CHUNK)
    batch_p = jnp.full((NPAD,), G, jnp.int32).at[:N].set(batch).reshape(1, NPAD)
    zeros_hh = jnp.zeros((NPAD, HH), jnp.float32)
    zeros_d = jnp.zeros((NPAD, DEGW), jnp.float32)
    ones_d = jnp.ones((CHUNK, DEGW), jnp.float32)

    degp = _sc_deg(dst_d, ones_d, zeros_d)
    dinv, hp2 = _tc_pre(degp, x, W1)

    for (Wn, b, g, be) in ((W2, b2, g2, be2), (W3, b3, g3, be3),
                           (W4, b4, g4, be4)):
        aggp = _sc_agg(src_a, dst_a, hp2, zeros_hh)
        hp2 = _tc_mid(aggp, hp2, dinv, b.reshape(1, H), g.reshape(1, H),
                      be.reshape(1, H), Wn)

    aggp = _sc_agg(src_a, dst_a, hp2, zeros_hh)
    out = _tc_fin(aggp, hp2, dinv, b4.reshape(1, H), g4.reshape(1, H),
                  be4.reshape(1, H), batch_p, fcW, fcb.reshape(1, C))
    return out


# speedup vs baseline: 30.5292x; 1.0001x over previous
import functools

import jax
import jax.numpy as jnp
from jax import lax
from jax.experimental import pallas as pl
from jax.experimental.pallas import tpu as pltpu
from jax.experimental.pallas import tpu_sc as plsc

N = 10000
E = 320000
F_IN = 128
H = 64
HH = H // 2
C = 10
G = 128
EPS = 1e-5

NPAD = 10112
ROWS_PER_TILE = NPAD // 16
CHUNK = 128
CPT_A = 160
CPG = 2
SLOTS = 4
LGS = SLOTS // 2
PAIRS = CPT_A // CPG
CPT_D = 80
EPAD = 16 * CPT_A * CHUNK
DEGW = 8

_mesh = plsc.VectorSubcoreMesh(core_axis_name="c", subcore_axis_name="s")


@functools.partial(
    pl.kernel,
    mesh=_mesh,
    out_type=jax.ShapeDtypeStruct((2, NPAD, HH), jnp.float32),
    scratch_types=[
        pltpu.VMEM((CPT_A, CHUNK), jnp.int32),
        pltpu.VMEM((CPT_A, CHUNK), jnp.int32),
        pltpu.VMEM((SLOTS, CPG * CHUNK, HH), jnp.float32),
        pltpu.VMEM_SHARED((NPAD, HH), jnp.float32),
        pltpu.VMEM_SHARED((NPAD, HH), jnp.float32),
    ] + [pltpu.SemaphoreType.DMA] * (2 * SLOTS),
    compiler_params=pltpu.CompilerParams(use_tc_tiling_on_sc=False),
)
def _sc_agg(src_hbm, dst_hbm, hp2_hbm, zeros_hbm, out_hbm,
            src_v, dst_v, rows, acc_sh, hp_sh, *sems):
    gsem = sems[:SLOTS]
    ssem = sems[SLOTS:]
    c = lax.axis_index("c")
    s = lax.axis_index("s")
    rsl = pl.ds(s * ROWS_PER_TILE, ROWS_PER_TILE)
    h_src = pltpu.async_copy(src_hbm.at[s], src_v, gsem[0])
    h_dst = pltpu.async_copy(dst_hbm.at[s], dst_v, gsem[1])
    h_hp = pltpu.async_copy(hp2_hbm.at[c, rsl], hp_sh.at[rsl], gsem[2])
    h_zero = pltpu.async_copy(zeros_hbm.at[rsl], acc_sh.at[rsl], gsem[3])
    h_src.wait()
    h_dst.wait()
    h_hp.wait()
    h_zero.wait()
    plsc.subcore_barrier()

    def _drain(sem, buf):
        pltpu.make_async_copy(hp2_hbm.at[0, pl.ds(0, CPG * CHUNK)], buf,
                              sem).wait()

    def _gathers(p, b):
        for q in range(CPG):
            pltpu.async_copy(hp_sh.at[src_v.at[p * CPG + q]],
                             rows.at[b, pl.ds(q * CHUNK, CHUNK)], gsem[b])

    def _scatters(p, b):
        for q in range(CPG):
            pltpu.async_copy(rows.at[b, pl.ds(q * CHUNK, CHUNK)],
                             acc_sh.at[dst_v.at[p * CPG + q]], ssem[b],
                             add=True)

    for b in range(LGS):
        _gathers(b, b)

    for b in range(SLOTS):
        _drain(gsem[b], rows.at[b])
        _scatters(b, b)
        bg = (b + LGS) % SLOTS
        if b >= LGS:
            _drain(ssem[bg], rows.at[bg])
        _gathers(b + LGS, bg)

    def round_body(jj, carry):
        for b in range(SLOTS):
            p = jj * SLOTS + b
            _drain(gsem[b], rows.at[b])
            _scatters(p, b)
            bg = (b + LGS) % SLOTS
            _drain(ssem[bg], rows.at[bg])

            @pl.when(p + LGS < PAIRS)
            def _():
                _gathers(p + LGS, bg)
        return carry

    lax.fori_loop(1, PAIRS // SLOTS, round_body, 0)
    for b in range(LGS, SLOTS):
        _drain(ssem[b], rows.at[b])
    plsc.subcore_barrier()
    pltpu.sync_copy(acc_sh.at[rsl], out_hbm.at[c, rsl])


@functools.partial(
    pl.kernel,
    mesh=_mesh,
    out_type=jax.ShapeDtypeStruct((2, NPAD, DEGW), jnp.float32),
    scratch_types=[
        pltpu.VMEM((CPT_D, CHUNK), jnp.int32),
        pltpu.VMEM((CHUNK, DEGW), jnp.float32),
        pltpu.VMEM_SHARED((NPAD, DEGW), jnp.float32),
        pltpu.SemaphoreType.DMA,
    ],
    compiler_params=pltpu.CompilerParams(use_tc_tiling_on_sc=False),
)
def _sc_deg(dst_hbm, ones_hbm, zeros_hbm, out_hbm, dst_v, ones_v, acc_sh,
            sem):
    c = lax.axis_index("c")
    s = lax.axis_index("s")
    wid = c * 16 + s
    pltpu.sync_copy(dst_hbm.at[wid], dst_v)
    pltpu.sync_copy(ones_hbm, ones_v)
    rsl = pl.ds(s * ROWS_PER_TILE, ROWS_PER_TILE)
    pltpu.sync_copy(zeros_hbm.at[rsl], acc_sh.at[rsl])
    plsc.subcore_barrier()

    def body(j, carry):
        pltpu.async_copy(ones_v, acc_sh.at[dst_v.at[j]], sem, add=True)
        return carry

    lax.fori_loop(0, CPT_D, body, 0)

    def drain(j, carry):
        pltpu.make_async_copy(ones_hbm, ones_v, sem).wait()
        return carry

    lax.fori_loop(0, CPT_D, drain, 0)
    plsc.subcore_barrier()
    pltpu.sync_copy(acc_sh.at[rsl], out_hbm.at[c, rsl])


def _row_mask():
    rows = lax.broadcasted_iota(jnp.int32, (NPAD, 1), 0)
    return (rows < N).astype(jnp.float32)


def _split_cols(h, out_ref):
    out_ref[0] = h[:, :HH]
    out_ref[1] = h[:, HH:]


def _tc_pre_body(degp_ref, x_ref, w1_ref, dinv_ref, hp2_ref):
    mask = _row_mask()
    deg = degp_ref[0, :, 0:1] + degp_ref[1, :, 0:1] + mask
    dinv = jnp.where(deg > 0.0, lax.rsqrt(jnp.maximum(deg, 1e-30)), 0.0)
    dinv_ref[...] = dinv
    h = jnp.dot(x_ref[...], w1_ref[...], preferred_element_type=jnp.float32)
    hp = h * dinv[:N]
    zpad = jnp.zeros((NPAD - N, HH), jnp.float32)
    hp2_ref[0] = jnp.concatenate([hp[:, :HH], zpad], axis=0)
    hp2_ref[1] = jnp.concatenate([hp[:, HH:], zpad], axis=0)


_tc_pre = pl.pallas_call(
    _tc_pre_body,
    out_shape=(
        jax.ShapeDtypeStruct((NPAD, 1), jnp.float32),
        jax.ShapeDtypeStruct((2, NPAD, HH), jnp.float32),
    ),
)


def _bn_relu(aggp_ref, hp2_ref, dinv_ref, b_ref, g_ref, be_ref):
    mask = _row_mask()
    dinv = dinv_ref[...]
    agg = jnp.concatenate([aggp_ref[0], aggp_ref[1]], axis=1)
    hp = jnp.concatenate([hp2_ref[0], hp2_ref[1]], axis=1)
    z = dinv * (agg + hp) + b_ref[...]
    mean = jnp.sum(z * mask, axis=0, keepdims=True) * (1.0 / N)
    zc = z - mean
    var = jnp.sum(mask * zc * zc, axis=0, keepdims=True) * (1.0 / N)
    zn = zc * lax.rsqrt(var + EPS)
    return jnp.maximum(g_ref[...] * zn + be_ref[...], 0.0) * mask


def _tc_mid_body(aggp_ref, hp2_ref, dinv_ref, b_ref, g_ref, be_ref, wn_ref,
                 hpn2_ref):
    a = _bn_relu(aggp_ref, hp2_ref, dinv_ref, b_ref, g_ref, be_ref)
    hn = jnp.dot(a, wn_ref[...], preferred_element_type=jnp.float32)
    _split_cols(hn * dinv_ref[...], hpn2_ref)


_tc_mid = pl.pallas_call(
    _tc_mid_body,
    out_shape=jax.ShapeDtypeStruct((2, NPAD, HH), jnp.float32),
)


def _tc_fin_body(aggp_ref, hp2_ref, dinv_ref, b_ref, g_ref, be_ref,
                 batch_ref, fcw_ref, fcb_ref, out_ref):
    a = _bn_relu(aggp_ref, hp2_ref, dinv_ref, b_ref, g_ref, be_ref)
    gids = lax.broadcasted_iota(jnp.int32, (G, NPAD), 0)
    onehot_t = (batch_ref[...] == gids).astype(jnp.float32)
    sums = jnp.dot(onehot_t, a, preferred_element_type=jnp.float32)
    counts = jnp.sum(onehot_t, axis=1, keepdims=True)
    pooled = sums / jnp.maximum(counts, 1.0)
    logits = jnp.dot(pooled, fcw_ref[...],
                     preferred_element_type=jnp.float32) + fcb_ref[...]
    m = jnp.max(logits, axis=-1, keepdims=True)
    lse = m + jnp.log(jnp.sum(jnp.exp(logits - m), axis=-1, keepdims=True))
    out_ref[...] = logits - lse


_tc_fin = pl.pallas_call(
    _tc_fin_body,
    out_shape=jax.ShapeDtypeStruct((G, C), jnp.float32),
)


def kernel(x, edge_index, batch, W1, b1, g1, be1, W2, b2, g2, be2,
           W3, b3, g3, be3, W4, b4, g4, be4, fcW, fcb):
    pad = jnp.full((max(0, EPAD - E),), N, jnp.int32)
    src_flat = jnp.concatenate([edge_index[0], pad])[:EPAD]
    dst_flat = jnp.concatenate([edge_index[1], pad])[:EPAD]
    src_a = src_flat.reshape(16, CPT_A, CHUNK)
    dst_a = dst_flat.reshape(16, CPT_A, CHUNK)
    dst_d = dst_flat.reshape(32, CPT_D, CHUNK)
    batch_p = jnp.full((NPAD,), G, jnp.int32).at[:N].set(batch).reshape(1, NPAD)
    zeros_hh = jnp.zeros((NPAD, HH), jnp.float32)
    zeros_d = jnp.zeros((NPAD, DEGW), jnp.float32)
    ones_d = jnp.ones((CHUNK, DEGW), jnp.float32)

    degp = _sc_deg(dst_d, ones_d, zeros_d)
    dinv, hp2 = _tc_pre(degp, x, W1)

    for (Wn, b, g, be) in ((W2, b2, g2, be2), (W3, b3, g3, be3),
                           (W4, b4, g4, be4)):
        aggp = _sc_agg(src_a, dst_a, hp2, zeros_hh)
        hp2 = _tc_mid(aggp, hp2, dinv, b.reshape(1, H), g.reshape(1, H),
                      be.reshape(1, H), Wn)

    aggp = _sc_agg(src_a, dst_a, hp2, zeros_hh)
    out = _tc_fin(aggp, hp2, dinv, b4.reshape(1, H), g4.reshape(1, H),
                  be4.reshape(1, H), batch_p, fcW, fcb.reshape(1, C))
    return out


# speedup vs baseline: 30.6023x; 1.0024x over previous
import functools

import jax
import jax.numpy as jnp
from jax import lax
from jax.experimental import pallas as pl
from jax.experimental.pallas import tpu as pltpu
from jax.experimental.pallas import tpu_sc as plsc

N = 10000
E = 320000
F_IN = 128
H = 64
HH = H // 2
C = 10
G = 128
EPS = 1e-5

NPAD = 10112
ROWS_PER_TILE = NPAD // 16
CHUNK = 128
CPT_A = 160
CPG = 2
SLOTS = 4
LGS = SLOTS // 2
PAIRS = CPT_A // CPG
CPT_D = 80
EPAD = 16 * CPT_A * CHUNK
DEGW = 8

_mesh = plsc.VectorSubcoreMesh(core_axis_name="c", subcore_axis_name="s")


@functools.partial(
    pl.kernel,
    mesh=_mesh,
    out_type=jax.ShapeDtypeStruct((2, NPAD, HH), jnp.float32),
    scratch_types=[
        pltpu.VMEM((CPT_A, CHUNK), jnp.int32),
        pltpu.VMEM((CPT_A, CHUNK), jnp.int32),
        pltpu.VMEM((SLOTS, CPG * CHUNK, HH), jnp.float32),
        pltpu.VMEM_SHARED((NPAD, HH), jnp.float32),
        pltpu.VMEM_SHARED((NPAD, HH), jnp.float32),
    ] + [pltpu.SemaphoreType.DMA] * (2 * SLOTS),
    compiler_params=pltpu.CompilerParams(use_tc_tiling_on_sc=False),
)
def _sc_agg(src_hbm, dst_hbm, hp2_hbm, zeros_hbm, out_hbm,
            src_v, dst_v, rows, acc_sh, hp_sh, *sems):
    gsem = sems[:SLOTS]
    ssem = sems[SLOTS:]
    c = lax.axis_index("c")
    s = lax.axis_index("s")
    rsl = pl.ds(s * ROWS_PER_TILE, ROWS_PER_TILE)
    h_src = pltpu.async_copy(src_hbm.at[s], src_v, gsem[0])
    h_dst = pltpu.async_copy(dst_hbm.at[s], dst_v, gsem[1])
    h_hp = pltpu.async_copy(hp2_hbm.at[c, rsl], hp_sh.at[rsl], gsem[2])
    h_zero = pltpu.async_copy(zeros_hbm.at[rsl], acc_sh.at[rsl], gsem[3])
    h_src.wait()
    h_dst.wait()
    h_hp.wait()
    h_zero.wait()
    plsc.subcore_barrier()

    def _drain(sem, buf):
        pltpu.make_async_copy(hp2_hbm.at[0, pl.ds(0, CPG * CHUNK)], buf,
                              sem).wait()

    def _gathers(p, b):
        for q in range(CPG):
            pltpu.async_copy(hp_sh.at[src_v.at[p * CPG + q]],
                             rows.at[b, pl.ds(q * CHUNK, CHUNK)], gsem[b])

    def _scatters(p, b):
        for q in range(CPG):
            pltpu.async_copy(rows.at[b, pl.ds(q * CHUNK, CHUNK)],
                             acc_sh.at[dst_v.at[p * CPG + q]], ssem[b],
                             add=True)

    for b in range(LGS):
        _gathers(b, b)

    for b in range(SLOTS):
        _drain(gsem[b], rows.at[b])
        _scatters(b, b)
        bg = (b + LGS) % SLOTS
        if b >= LGS:
            _drain(ssem[bg], rows.at[bg])
        _gathers(b + LGS, bg)

    def round_body(jj, carry):
        for b in range(SLOTS):
            p = jj * SLOTS + b
            _drain(gsem[b], rows.at[b])
            _scatters(p, b)
            bg = (b + LGS) % SLOTS
            _drain(ssem[bg], rows.at[bg])

            @pl.when(p + LGS < PAIRS)
            def _():
                _gathers(p + LGS, bg)
        return carry

    lax.fori_loop(1, PAIRS // SLOTS, round_body, 0)
    for b in range(LGS, SLOTS):
        _drain(ssem[b], rows.at[b])
    plsc.subcore_barrier()
    pltpu.sync_copy(acc_sh.at[rsl], out_hbm.at[c, rsl])


@functools.partial(
    pl.kernel,
    mesh=_mesh,
    out_type=jax.ShapeDtypeStruct((2, NPAD, DEGW), jnp.float32),
    scratch_types=[
        pltpu.VMEM((CPT_D, CHUNK), jnp.int32),
        pltpu.VMEM((CHUNK, DEGW), jnp.float32),
        pltpu.VMEM_SHARED((NPAD, DEGW), jnp.float32),
        pltpu.SemaphoreType.DMA,
    ],
    compiler_params=pltpu.CompilerParams(use_tc_tiling_on_sc=False),
)
def _sc_deg(dst_hbm, ones_hbm, zeros_hbm, out_hbm, dst_v, ones_v, acc_sh,
            sem):
    c = lax.axis_index("c")
    s = lax.axis_index("s")
    wid = c * 16 + s
    rsl = pl.ds(s * ROWS_PER_TILE, ROWS_PER_TILE)
    h_dst = pltpu.async_copy(dst_hbm.at[wid], dst_v, sem)
    h_ones = pltpu.async_copy(ones_hbm, ones_v, sem)
    h_zero = pltpu.async_copy(zeros_hbm.at[rsl], acc_sh.at[rsl], sem)
    h_dst.wait()
    h_ones.wait()
    h_zero.wait()
    plsc.subcore_barrier()

    def body(j, carry):
        pltpu.async_copy(ones_v, acc_sh.at[dst_v.at[j]], sem, add=True)
        return carry

    lax.fori_loop(0, CPT_D, body, 0)

    def drain(j, carry):
        pltpu.make_async_copy(ones_hbm, ones_v, sem).wait()
        return carry

    lax.fori_loop(0, CPT_D, drain, 0)
    plsc.subcore_barrier()
    pltpu.sync_copy(acc_sh.at[rsl], out_hbm.at[c, rsl])


def _row_mask():
    rows = lax.broadcasted_iota(jnp.int32, (NPAD, 1), 0)
    return (rows < N).astype(jnp.float32)


def _split_cols(h, out_ref):
    out_ref[0] = h[:, :HH]
    out_ref[1] = h[:, HH:]


def _tc_pre_body(degp_ref, x_ref, w1_ref, dinv_ref, hp2_ref):
    mask = _row_mask()
    deg = degp_ref[0, :, 0:1] + degp_ref[1, :, 0:1] + mask
    dinv = jnp.where(deg > 0.0, lax.rsqrt(jnp.maximum(deg, 1e-30)), 0.0)
    dinv_ref[...] = dinv
    h = jnp.dot(x_ref[...], w1_ref[...], preferred_element_type=jnp.float32)
    hp = h * dinv[:N]
    zpad = jnp.zeros((NPAD - N, HH), jnp.float32)
    hp2_ref[0] = jnp.concatenate([hp[:, :HH], zpad], axis=0)
    hp2_ref[1] = jnp.concatenate([hp[:, HH:], zpad], axis=0)


_tc_pre = pl.pallas_call(
    _tc_pre_body,
    out_shape=(
        jax.ShapeDtypeStruct((NPAD, 1), jnp.float32),
        jax.ShapeDtypeStruct((2, NPAD, HH), jnp.float32),
    ),
)


def _bn_relu(aggp_ref, hp2_ref, dinv_ref, b_ref, g_ref, be_ref):
    mask = _row_mask()
    dinv = dinv_ref[...]
    agg = jnp.concatenate([aggp_ref[0], aggp_ref[1]], axis=1)
    hp = jnp.concatenate([hp2_ref[0], hp2_ref[1]], axis=1)
    z = dinv * (agg + hp) + b_ref[...]
    mean = jnp.sum(z * mask, axis=0, keepdims=True) * (1.0 / N)
    zc = z - mean
    var = jnp.sum(mask * zc * zc, axis=0, keepdims=True) * (1.0 / N)
    zn = zc * lax.rsqrt(var + EPS)
    return jnp.maximum(g_ref[...] * zn + be_ref[...], 0.0) * mask


def _tc_mid_body(aggp_ref, hp2_ref, dinv_ref, b_ref, g_ref, be_ref, wn_ref,
                 hpn2_ref):
    a = _bn_relu(aggp_ref, hp2_ref, dinv_ref, b_ref, g_ref, be_ref)
    hn = jnp.dot(a, wn_ref[...], preferred_element_type=jnp.float32)
    _split_cols(hn * dinv_ref[...], hpn2_ref)


_tc_mid = pl.pallas_call(
    _tc_mid_body,
    out_shape=jax.ShapeDtypeStruct((2, NPAD, HH), jnp.float32),
)


def _tc_fin_body(aggp_ref, hp2_ref, dinv_ref, b_ref, g_ref, be_ref,
                 batch_ref, fcw_ref, fcb_ref, out_ref):
    a = _bn_relu(aggp_ref, hp2_ref, dinv_ref, b_ref, g_ref, be_ref)
    gids = lax.broadcasted_iota(jnp.int32, (G, NPAD), 0)
    onehot_t = (batch_ref[...] == gids).astype(jnp.float32)
    sums = jnp.dot(onehot_t, a, preferred_element_type=jnp.float32)
    counts = jnp.sum(onehot_t, axis=1, keepdims=True)
    pooled = sums / jnp.maximum(counts, 1.0)
    logits = jnp.dot(pooled, fcw_ref[...],
                     preferred_element_type=jnp.float32) + fcb_ref[...]
    m = jnp.max(logits, axis=-1, keepdims=True)
    lse = m + jnp.log(jnp.sum(jnp.exp(logits - m), axis=-1, keepdims=True))
    out_ref[...] = logits - lse


_tc_fin = pl.pallas_call(
    _tc_fin_body,
    out_shape=jax.ShapeDtypeStruct((G, C), jnp.float32),
)


def kernel(x, edge_index, batch, W1, b1, g1, be1, W2, b2, g2, be2,
           W3, b3, g3, be3, W4, b4, g4, be4, fcW, fcb):
    pad = jnp.full((max(0, EPAD - E),), N, jnp.int32)
    src_flat = jnp.concatenate([edge_index[0], pad])[:EPAD]
    dst_flat = jnp.concatenate([edge_index[1], pad])[:EPAD]
    src_a = src_flat.reshape(16, CPT_A, CHUNK)
    dst_a = dst_flat.reshape(16, CPT_A, CHUNK)
    dst_d = dst_flat.reshape(32, CPT_D, CHUNK)
    batch_p = jnp.full((NPAD,), G, jnp.int32).at[:N].set(batch).reshape(1, NPAD)
    zeros_hh = jnp.zeros((NPAD, HH), jnp.float32)
    zeros_d = jnp.zeros((NPAD, DEGW), jnp.float32)
    ones_d = jnp.ones((CHUNK, DEGW), jnp.float32)

    degp = _sc_deg(dst_d, ones_d, zeros_d)
    dinv, hp2 = _tc_pre(degp, x, W1)

    for (Wn, b, g, be) in ((W2, b2, g2, be2), (W3, b3, g3, be3),
                           (W4, b4, g4, be4)):
        aggp = _sc_agg(src_a, dst_a, hp2, zeros_hh)
        hp2 = _tc_mid(aggp, hp2, dinv, b.reshape(1, H), g.reshape(1, H),
                      be.reshape(1, H), Wn)

    aggp = _sc_agg(src_a, dst_a, hp2, zeros_hh)
    out = _tc_fin(aggp, hp2, dinv, b4.reshape(1, H), g4.reshape(1, H),
                  be4.reshape(1, H), batch_p, fcW, fcb.reshape(1, C))
    return out
